# Initial kernel scaffold; baseline (speedup 1.0000x reference)
import jax
import jax.numpy as jnp
from jax.experimental import pallas as pl


def kernel(x, edge_index, W1, b1, W2, b2, lin_W, lin_b):
    raise NotImplementedError("write your pallas kernel here")


# speedup vs baseline: 10.5614x; 10.5614x over previous
import functools

import jax
import jax.numpy as jnp
from jax import lax
from jax.experimental import pallas as pl
from jax.experimental.pallas import tpu as pltpu
from jax.experimental.pallas import tpu_sc as plsc

N = 10000
E = 320000
D = 128
NCLS = 40
NCLS_PAD = 64

NC = 2
NS = 16
NW = NC * NS
CH = 128
CPW = 80
NCHUNK = NW * CPW
EPAD = NCHUNK * CH
NP = 10240
RPT = NP // NS
DUMMY = 10100
DW = 128
IB = 16


def _zero_fill(buf, ncols, value=0.0):
  v = jnp.full((16,), value, dtype=jnp.float32)

  def row(i, carry):
    for g in range(ncols // 16):
      buf[i, pl.ds(g * 16, 16)] = v
    return carry

  lax.fori_loop(0, CH, row, 0)


def _deg_body(dst_hbm, out_hbm, acc_sh, dst_v, ones_v):
  c = lax.axis_index("c")
  s = lax.axis_index("s")
  wid = s * NC + c

  pltpu.sync_copy(dst_hbm.at[pl.ds(wid * CPW, CPW)], dst_v)
  _zero_fill(ones_v, DW, 0.0)
  for k in range(RPT // CH):
    pltpu.sync_copy(ones_v, acc_sh.at[pl.ds(s * RPT + k * CH, CH)])
  _zero_fill(ones_v, DW, 1.0)
  plsc.subcore_barrier()

  def body(j, carry):
    pltpu.sync_copy(ones_v, acc_sh.at[dst_v.at[j]], add=True)
    return carry

  lax.fori_loop(0, CPW, body, 0)
  plsc.subcore_barrier()
  for k in range(RPT // CH):
    r = s * RPT + k * CH
    pltpu.sync_copy(acc_sh.at[pl.ds(r, CH)], out_hbm.at[c].at[pl.ds(r, CH)])


def _agg_body(hp_hbm, src_hbm, dst_hbm, out_hbm,
              acc_sh, src_v, dst_v, gbuf0, gbuf1, sem0, sem1):
  c = lax.axis_index("c")
  s = lax.axis_index("s")
  wid = s * NC + c

  _zero_fill(gbuf0, D, 0.0)
  for k in range(RPT // CH):
    pltpu.sync_copy(gbuf0, acc_sh.at[pl.ds(s * RPT + k * CH, CH)])
  plsc.subcore_barrier()

  bufs = (gbuf0, gbuf1)
  sems = (sem0, sem1)

  def blk_body(bi, carry):
    base = wid * CPW + bi * IB
    pltpu.sync_copy(src_hbm.at[pl.ds(base, IB)], src_v)
    pltpu.sync_copy(dst_hbm.at[pl.ds(base, IB)], dst_v)
    descs = [
        pltpu.async_copy(hp_hbm.at[src_v.at[0]], gbuf0, sem0),
        pltpu.async_copy(hp_hbm.at[src_v.at[1]], gbuf1, sem1),
    ]
    for j in range(IB):
      b = j % 2
      descs[b].wait()
      pltpu.sync_copy(bufs[b], acc_sh.at[dst_v.at[j]], add=True)
      if j + 2 < IB:
        descs[b] = pltpu.async_copy(
            hp_hbm.at[src_v.at[j + 2]], bufs[b], sems[b])
    return carry

  lax.fori_loop(0, CPW // IB, blk_body, 0)
  plsc.subcore_barrier()
  for k in range(RPT // CH):
    r = s * RPT + k * CH
    pltpu.sync_copy(acc_sh.at[pl.ds(r, CH)], out_hbm.at[c].at[pl.ds(r, CH)])


@functools.cache
def _sc_kernels():
  mesh = plsc.VectorSubcoreMesh(
      core_axis_name="c", subcore_axis_name="s", num_cores=NC, num_subcores=NS
  )
  deg_kernel = pl.kernel(
      _deg_body,
      out_type=jax.ShapeDtypeStruct((NC, NP, DW), jnp.float32),
      mesh=mesh,
      scratch_types=[
          pltpu.VMEM_SHARED((NP, DW), jnp.float32),
          pltpu.VMEM((CPW, CH), jnp.int32),
          pltpu.VMEM((CH, DW), jnp.float32),
      ],
  )
  agg_kernel = pl.kernel(
      _agg_body,
      out_type=jax.ShapeDtypeStruct((NC, NP, D), jnp.float32),
      mesh=mesh,
      scratch_types=[
          pltpu.VMEM_SHARED((NP, D), jnp.float32),
          pltpu.VMEM((IB, CH), jnp.int32),
          pltpu.VMEM((IB, CH), jnp.int32),
          pltpu.VMEM((CH, D), jnp.float32),
          pltpu.VMEM((CH, D), jnp.float32),
          pltpu.SemaphoreType.DMA,
          pltpu.SemaphoreType.DMA,
      ],
  )
  return deg_kernel, agg_kernel


_RB = 512
_GRID = NP // _RB


def _dinv_of(deg_ref):
  deg = deg_ref[0, :, 0:1] + deg_ref[1, :, 0:1] + 1.0
  return lax.rsqrt(deg)


def _tc1_body(x_ref, w1_ref, deg_ref, out_ref):
  dinv = _dinv_of(deg_ref)
  h = jnp.dot(x_ref[...], w1_ref[...], preferred_element_type=jnp.float32)
  out_ref[...] = h * dinv


def _tc2_body(acc_ref, hp_ref, deg_ref, w2_ref, b1_ref, out_ref):
  dinv = _dinv_of(deg_ref)
  agg = acc_ref[0] + acc_ref[1] + hp_ref[...]
  z = jnp.maximum(agg * dinv + b1_ref[...], 0.0)
  h = jnp.dot(z, w2_ref[...], preferred_element_type=jnp.float32)
  out_ref[...] = h * dinv


def _tc3_body(acc_ref, hp_ref, deg_ref, b2_ref, lw_ref, lb_ref, out_ref):
  dinv = _dinv_of(deg_ref)
  agg = acc_ref[0] + acc_ref[1] + hp_ref[...]
  z = jnp.maximum(agg * dinv + b2_ref[...], 0.0)
  logits = jnp.dot(z, lw_ref[...], preferred_element_type=jnp.float32)
  logits = logits + lb_ref[...]
  m = jnp.max(logits, axis=1, keepdims=True)
  e = jnp.exp(logits - m)
  lse = jnp.log(jnp.sum(e, axis=1, keepdims=True)) + m
  out_ref[...] = logits - lse


def _row_spec():
  return pl.BlockSpec((_RB, D), lambda i: (i, 0))


def _acc_spec():
  return pl.BlockSpec((NC, _RB, D), lambda i: (0, i, 0))


def _deg_spec():
  return pl.BlockSpec((NC, _RB, DW), lambda i: (0, i, 0))


def _full(shape):
  return pl.BlockSpec(shape, lambda i: tuple(0 for _ in shape))


_tc1 = pl.pallas_call(
    _tc1_body,
    grid=(_GRID,),
    in_specs=[_row_spec(), _full((D, D)), _deg_spec()],
    out_specs=_row_spec(),
    out_shape=jax.ShapeDtypeStruct((NP, D), jnp.float32),
)

_tc2 = pl.pallas_call(
    _tc2_body,
    grid=(_GRID,),
    in_specs=[_acc_spec(), _row_spec(), _deg_spec(), _full((D, D)),
              _full((1, D))],
    out_specs=_row_spec(),
    out_shape=jax.ShapeDtypeStruct((NP, D), jnp.float32),
)

_tc3 = pl.pallas_call(
    _tc3_body,
    grid=(_GRID,),
    in_specs=[_acc_spec(), _row_spec(), _deg_spec(), _full((1, D)),
              _full((D, NCLS_PAD)), _full((1, NCLS_PAD))],
    out_specs=pl.BlockSpec((_RB, NCLS_PAD), lambda i: (i, 0)),
    out_shape=jax.ShapeDtypeStruct((NP, NCLS_PAD), jnp.float32),
)


@jax.jit
def kernel(x, edge_index, W1, b1, W2, b2, lin_W, lin_b):
  x_pad = jnp.concatenate(
      [x, jnp.zeros((NP - N, D), dtype=x.dtype)], axis=0)
  pad = jnp.full((EPAD - E,), DUMMY, dtype=jnp.int32)
  src2d = jnp.concatenate([edge_index[0], pad]).reshape(NCHUNK, CH)
  dst2d = jnp.concatenate([edge_index[1], pad]).reshape(NCHUNK, CH)
  b1r = b1.reshape(1, D)
  b2r = b2.reshape(1, D)
  lw = jnp.concatenate(
      [lin_W, jnp.zeros((D, NCLS_PAD - NCLS), dtype=lin_W.dtype)], axis=1)
  lb = jnp.concatenate(
      [lin_b, jnp.full((NCLS_PAD - NCLS,), -1e30, dtype=lin_b.dtype)]
  ).reshape(1, NCLS_PAD)

  deg_kernel, agg_kernel = _sc_kernels()
  deg2 = deg_kernel(dst2d)
  hp1 = _tc1(x_pad, W1, deg2)
  acc1 = agg_kernel(hp1, src2d, dst2d)
  hp2 = _tc2(acc1, hp1, deg2, W2, b1r)
  acc2 = agg_kernel(hp2, src2d, dst2d)
  out = _tc3(acc2, hp2, deg2, b2r, lw, lb)
  return out[:N, :NCLS]


# speedup vs baseline: 25.1392x; 2.3803x over previous
import functools

import jax
import jax.numpy as jnp
from jax import lax
from jax.experimental import pallas as pl
from jax.experimental.pallas import tpu as pltpu
from jax.experimental.pallas import tpu_sc as plsc

N = 10000
E = 320000
D = 128
NCLS = 40
NCLS_PAD = 64

NC = 2
NS = 16
NW = NC * NS
CH = 128
CPW = 80
NCHUNK = NW * CPW
EPAD = NCHUNK * CH
NP = 10240
RPT = NP // NS
DUMMY = 10100
DW = 128
IB = 16


def _zero_fill(buf, ncols, value=0.0):
  v = jnp.full((16,), value, dtype=jnp.float32)

  def row(i, carry):
    for g in range(ncols // 16):
      buf[i, pl.ds(g * 16, 16)] = v
    return carry

  lax.fori_loop(0, CH, row, 0)


def _deg_body(dst_hbm, out_hbm, acc_sh, dst_v, ones_v):
  c = lax.axis_index("c")
  s = lax.axis_index("s")
  wid = s * NC + c

  pltpu.sync_copy(dst_hbm.at[pl.ds(wid * CPW, CPW)], dst_v)
  _zero_fill(ones_v, DW, 0.0)
  for k in range(RPT // CH):
    pltpu.sync_copy(ones_v, acc_sh.at[pl.ds(s * RPT + k * CH, CH)])
  _zero_fill(ones_v, DW, 1.0)
  plsc.subcore_barrier()

  def body(j, carry):
    pltpu.sync_copy(ones_v, acc_sh.at[dst_v.at[j]], add=True)
    return carry

  lax.fori_loop(0, CPW, body, 0)
  plsc.subcore_barrier()
  for k in range(RPT // CH):
    r = s * RPT + k * CH
    pltpu.sync_copy(acc_sh.at[pl.ds(r, CH)], out_hbm.at[c].at[pl.ds(r, CH)])


def _agg_body(hp_hbm, src_hbm, dst_hbm, out_hbm,
              acc_sh, src_v, dst_v, gbuf0, gbuf1, sem0, sem1):
  c = lax.axis_index("c")
  s = lax.axis_index("s")
  wid = s * NC + c

  _zero_fill(gbuf0, D, 0.0)
  for k in range(RPT // CH):
    pltpu.sync_copy(gbuf0, acc_sh.at[pl.ds(s * RPT + k * CH, CH)])
  plsc.subcore_barrier()

  bufs = (gbuf0, gbuf1)
  sems = (sem0, sem1)

  def blk_body(bi, carry):
    base = wid * CPW + bi * IB
    pltpu.sync_copy(src_hbm.at[pl.ds(base, IB)], src_v)
    pltpu.sync_copy(dst_hbm.at[pl.ds(base, IB)], dst_v)
    descs = [
        pltpu.async_copy(hp_hbm.at[src_v.at[0]], gbuf0, sem0),
        pltpu.async_copy(hp_hbm.at[src_v.at[1]], gbuf1, sem1),
    ]
    for j in range(IB):
      b = j % 2
      descs[b].wait()
      pltpu.sync_copy(bufs[b], acc_sh.at[dst_v.at[j]], add=True)
      if j + 2 < IB:
        descs[b] = pltpu.async_copy(
            hp_hbm.at[src_v.at[j + 2]], bufs[b], sems[b])
    return carry

  lax.fori_loop(0, CPW // IB, blk_body, 0)
  plsc.subcore_barrier()
  for k in range(RPT // CH):
    r = s * RPT + k * CH
    pltpu.sync_copy(acc_sh.at[pl.ds(r, CH)], out_hbm.at[c].at[pl.ds(r, CH)])


@functools.cache
def _sc_kernels():
  mesh = plsc.VectorSubcoreMesh(
      core_axis_name="c", subcore_axis_name="s", num_cores=NC, num_subcores=NS
  )
  deg_kernel = pl.kernel(
      _deg_body,
      out_type=jax.ShapeDtypeStruct((NC, NP, DW), jnp.float32),
      mesh=mesh,
      scratch_types=[
          pltpu.VMEM_SHARED((NP, DW), jnp.float32),
          pltpu.VMEM((CPW, CH), jnp.int32),
          pltpu.VMEM((CH, DW), jnp.float32),
      ],
  )
  agg_kernel = pl.kernel(
      _agg_body,
      out_type=jax.ShapeDtypeStruct((NC, NP, D), jnp.float32),
      mesh=mesh,
      scratch_types=[
          pltpu.VMEM_SHARED((NP, D), jnp.float32),
          pltpu.VMEM((IB, CH), jnp.int32),
          pltpu.VMEM((IB, CH), jnp.int32),
          pltpu.VMEM((CH, D), jnp.float32),
          pltpu.VMEM((CH, D), jnp.float32),
          pltpu.SemaphoreType.DMA,
          pltpu.SemaphoreType.DMA,
      ],
  )
  return deg_kernel, agg_kernel


_RB = 512
_GRID = NP // _RB


def _dinv_of(deg_ref):
  deg = deg_ref[0, :, 0:1] + deg_ref[1, :, 0:1] + 1.0
  return lax.rsqrt(deg)


def _tc1_body(x_ref, w1_ref, deg_ref, out_ref):
  dinv = _dinv_of(deg_ref)
  h = jnp.dot(x_ref[...], w1_ref[...], preferred_element_type=jnp.float32)
  out_ref[...] = h * dinv


def _tc2_body(acc_ref, hp_ref, deg_ref, w2_ref, b1_ref, out_ref):
  dinv = _dinv_of(deg_ref)
  agg = acc_ref[0] + acc_ref[1] + hp_ref[...]
  z = jnp.maximum(agg * dinv + b1_ref[...], 0.0)
  h = jnp.dot(z, w2_ref[...], preferred_element_type=jnp.float32)
  out_ref[...] = h * dinv


def _tc3_body(acc_ref, hp_ref, deg_ref, b2_ref, lw_ref, lb_ref, out_ref):
  dinv = _dinv_of(deg_ref)
  agg = acc_ref[0] + acc_ref[1] + hp_ref[...]
  z = jnp.maximum(agg * dinv + b2_ref[...], 0.0)
  logits = jnp.dot(z, lw_ref[...], preferred_element_type=jnp.float32)
  logits = logits + lb_ref[...]
  m = jnp.max(logits, axis=1, keepdims=True)
  e = jnp.exp(logits - m)
  lse = jnp.log(jnp.sum(e, axis=1, keepdims=True)) + m
  out_ref[...] = logits - lse


def _row_spec():
  return pl.BlockSpec((_RB, D), lambda i: (i, 0))


def _acc_spec():
  return pl.BlockSpec((NC, _RB, D), lambda i: (0, i, 0))


def _deg_spec():
  return pl.BlockSpec((NC, _RB, DW), lambda i: (0, i, 0))


def _full(shape):
  return pl.BlockSpec(shape, lambda i: tuple(0 for _ in shape))


_tc1 = pl.pallas_call(
    _tc1_body,
    grid=(_GRID,),
    in_specs=[_row_spec(), _full((D, D)), _deg_spec()],
    out_specs=_row_spec(),
    out_shape=jax.ShapeDtypeStruct((NP, D), jnp.float32),
)

_tc2 = pl.pallas_call(
    _tc2_body,
    grid=(_GRID,),
    in_specs=[_acc_spec(), _row_spec(), _deg_spec(), _full((D, D)),
              _full((1, D))],
    out_specs=_row_spec(),
    out_shape=jax.ShapeDtypeStruct((NP, D), jnp.float32),
)

_tc3 = pl.pallas_call(
    _tc3_body,
    grid=(_GRID,),
    in_specs=[_acc_spec(), _row_spec(), _deg_spec(), _full((1, D)),
              _full((D, NCLS_PAD)), _full((1, NCLS_PAD))],
    out_specs=pl.BlockSpec((_RB, NCLS_PAD), lambda i: (i, 0)),
    out_shape=jax.ShapeDtypeStruct((NP, NCLS_PAD), jnp.float32),
)


@jax.jit
def kernel(x, edge_index, W1, b1, W2, b2, lin_W, lin_b):
  x_pad = jnp.concatenate(
      [x, jnp.zeros((NP - N, D), dtype=x.dtype)], axis=0)
  pad = N + (jnp.arange(EPAD - E, dtype=jnp.int32) % (NP - N))
  src2d = jnp.concatenate([edge_index[0], pad]).reshape(NCHUNK, CH)
  dst2d = jnp.concatenate([edge_index[1], pad]).reshape(NCHUNK, CH)
  b1r = b1.reshape(1, D)
  b2r = b2.reshape(1, D)
  lw = jnp.concatenate(
      [lin_W, jnp.zeros((D, NCLS_PAD - NCLS), dtype=lin_W.dtype)], axis=1)
  lb = jnp.concatenate(
      [lin_b, jnp.full((NCLS_PAD - NCLS,), -1e30, dtype=lin_b.dtype)]
  ).reshape(1, NCLS_PAD)

  deg_kernel, agg_kernel = _sc_kernels()
  deg2 = deg_kernel(dst2d)
  hp1 = _tc1(x_pad, W1, deg2)
  acc1 = agg_kernel(hp1, src2d, dst2d)
  hp2 = _tc2(acc1, hp1, deg2, W2, b1r)
  acc2 = agg_kernel(hp2, src2d, dst2d)
  out = _tc3(acc2, hp2, deg2, b2r, lw, lb)
  return out[:N, :NCLS]


# speedup vs baseline: 25.4579x; 1.0127x over previous
import functools

import jax
import jax.numpy as jnp
from jax import lax
from jax.experimental import pallas as pl
from jax.experimental.pallas import tpu as pltpu
from jax.experimental.pallas import tpu_sc as plsc

N = 10000
E = 320000
D = 128
NCLS = 40
NCLS_PAD = 64

NC = 2
NS = 16
NW = NC * NS
CH = 128
CPW = 80
NCHUNK = NW * CPW
EPAD = NCHUNK * CH
NP = 10240
RPT = NP // NS
DUMMY = 10100
DW = 128
IB = 16

CHA = 64
NBUF = 4
CPTA = EPAD // (NW * CHA)
NCHUNKA = NW * CPTA
NBLK = CPTA // IB


def _zero_fill(buf, ncols, value=0.0):
  v = jnp.full((16,), value, dtype=jnp.float32)

  def row(i, carry):
    for g in range(ncols // 16):
      buf[i, pl.ds(g * 16, 16)] = v
    return carry

  lax.fori_loop(0, buf.shape[0], row, 0)


def _deg_body(dst_hbm, out_hbm, acc_sh, dst_v, ones_v):
  c = lax.axis_index("c")
  s = lax.axis_index("s")
  wid = s * NC + c

  pltpu.sync_copy(dst_hbm.at[pl.ds(wid * CPW, CPW)], dst_v)
  _zero_fill(ones_v, DW, 0.0)
  for k in range(RPT // CH):
    pltpu.sync_copy(ones_v, acc_sh.at[pl.ds(s * RPT + k * CH, CH)])
  _zero_fill(ones_v, DW, 1.0)
  plsc.subcore_barrier()

  def body(j, carry):
    pltpu.sync_copy(ones_v, acc_sh.at[dst_v.at[j]], add=True)
    return carry

  lax.fori_loop(0, CPW, body, 0)
  plsc.subcore_barrier()
  for k in range(RPT // CH):
    r = s * RPT + k * CH
    pltpu.sync_copy(acc_sh.at[pl.ds(r, CH)], out_hbm.at[c].at[pl.ds(r, CH)])


def _agg_body(hp_hbm, src_hbm, dst_hbm, out_hbm,
              acc_sh, src_v, dst_v, bufs, sems):
  c = lax.axis_index("c")
  s = lax.axis_index("s")
  wid = s * NC + c

  _zero_fill(bufs[0], D, 0.0)
  for k in range(RPT // CHA):
    pltpu.sync_copy(bufs[0], acc_sh.at[pl.ds(s * RPT + k * CHA, CHA)])
  plsc.subcore_barrier()

  def blk_body(bi, carry):
    base = wid * CPTA + bi * IB
    pltpu.sync_copy(src_hbm.at[pl.ds(base, IB)], src_v)
    pltpu.sync_copy(dst_hbm.at[pl.ds(base, IB)], dst_v)
    descs = [
        pltpu.async_copy(hp_hbm.at[src_v.at[j]], bufs[j], sems[j])
        for j in range(NBUF - 1)
    ]
    descs.append(None)
    for j in range(IB):
      b = j % NBUF
      descs[b].wait()
      if j + NBUF - 1 < IB:
        bn = (j + NBUF - 1) % NBUF
        descs[bn] = pltpu.async_copy(
            hp_hbm.at[src_v.at[j + NBUF - 1]], bufs[bn], sems[bn])
      pltpu.sync_copy(bufs[b], acc_sh.at[dst_v.at[j]], add=True)
    return carry

  lax.fori_loop(0, NBLK, blk_body, 0)
  plsc.subcore_barrier()
  for k in range(RPT // CH):
    r = s * RPT + k * CH
    pltpu.sync_copy(acc_sh.at[pl.ds(r, CH)], out_hbm.at[c].at[pl.ds(r, CH)])


@functools.cache
def _sc_kernels():
  mesh = plsc.VectorSubcoreMesh(
      core_axis_name="c", subcore_axis_name="s", num_cores=NC, num_subcores=NS
  )
  deg_kernel = pl.kernel(
      _deg_body,
      out_type=jax.ShapeDtypeStruct((NC, NP, DW), jnp.float32),
      mesh=mesh,
      scratch_types=[
          pltpu.VMEM_SHARED((NP, DW), jnp.float32),
          pltpu.VMEM((CPW, CH), jnp.int32),
          pltpu.VMEM((CH, DW), jnp.float32),
      ],
  )
  agg_kernel = pl.kernel(
      _agg_body,
      out_type=jax.ShapeDtypeStruct((NC, NP, D), jnp.float32),
      mesh=mesh,
      scratch_types=[
          pltpu.VMEM_SHARED((NP, D), jnp.float32),
          pltpu.VMEM((IB, CHA), jnp.int32),
          pltpu.VMEM((IB, CHA), jnp.int32),
          [pltpu.VMEM((CHA, D), jnp.float32) for _ in range(NBUF)],
          [pltpu.SemaphoreType.DMA for _ in range(NBUF)],
      ],
  )
  return deg_kernel, agg_kernel


_RB = 512
_GRID = NP // _RB


def _dinv_of(deg_ref):
  deg = deg_ref[0, :, 0:1] + deg_ref[1, :, 0:1] + 1.0
  return lax.rsqrt(deg)


def _tc1_body(x_ref, w1_ref, deg_ref, out_ref):
  dinv = _dinv_of(deg_ref)
  h = jnp.dot(x_ref[...], w1_ref[...], preferred_element_type=jnp.float32)
  out_ref[...] = h * dinv


def _tc2_body(acc_ref, hp_ref, deg_ref, w2_ref, b1_ref, out_ref):
  dinv = _dinv_of(deg_ref)
  agg = acc_ref[0] + acc_ref[1] + hp_ref[...]
  z = jnp.maximum(agg * dinv + b1_ref[...], 0.0)
  h = jnp.dot(z, w2_ref[...], preferred_element_type=jnp.float32)
  out_ref[...] = h * dinv


def _tc3_body(acc_ref, hp_ref, deg_ref, b2_ref, lw_ref, lb_ref, out_ref):
  dinv = _dinv_of(deg_ref)
  agg = acc_ref[0] + acc_ref[1] + hp_ref[...]
  z = jnp.maximum(agg * dinv + b2_ref[...], 0.0)
  logits = jnp.dot(z, lw_ref[...], preferred_element_type=jnp.float32)
  logits = logits + lb_ref[...]
  m = jnp.max(logits, axis=1, keepdims=True)
  e = jnp.exp(logits - m)
  lse = jnp.log(jnp.sum(e, axis=1, keepdims=True)) + m
  out_ref[...] = logits - lse


def _row_spec():
  return pl.BlockSpec((_RB, D), lambda i: (i, 0))


def _acc_spec():
  return pl.BlockSpec((NC, _RB, D), lambda i: (0, i, 0))


def _deg_spec():
  return pl.BlockSpec((NC, _RB, DW), lambda i: (0, i, 0))


def _full(shape):
  return pl.BlockSpec(shape, lambda i: tuple(0 for _ in shape))


_tc1 = pl.pallas_call(
    _tc1_body,
    grid=(_GRID,),
    in_specs=[_row_spec(), _full((D, D)), _deg_spec()],
    out_specs=_row_spec(),
    out_shape=jax.ShapeDtypeStruct((NP, D), jnp.float32),
)

_tc2 = pl.pallas_call(
    _tc2_body,
    grid=(_GRID,),
    in_specs=[_acc_spec(), _row_spec(), _deg_spec(), _full((D, D)),
              _full((1, D))],
    out_specs=_row_spec(),
    out_shape=jax.ShapeDtypeStruct((NP, D), jnp.float32),
)

_tc3 = pl.pallas_call(
    _tc3_body,
    grid=(_GRID,),
    in_specs=[_acc_spec(), _row_spec(), _deg_spec(), _full((1, D)),
              _full((D, NCLS_PAD)), _full((1, NCLS_PAD))],
    out_specs=pl.BlockSpec((_RB, NCLS_PAD), lambda i: (i, 0)),
    out_shape=jax.ShapeDtypeStruct((NP, NCLS_PAD), jnp.float32),
)


@jax.jit
def kernel(x, edge_index, W1, b1, W2, b2, lin_W, lin_b):
  x_pad = jnp.concatenate(
      [x, jnp.zeros((NP - N, D), dtype=x.dtype)], axis=0)
  pad = N + (jnp.arange(EPAD - E, dtype=jnp.int32) % (NP - N))
  src_flat = jnp.concatenate([edge_index[0], pad])
  dst_flat = jnp.concatenate([edge_index[1], pad])
  src2da = src_flat.reshape(NCHUNKA, CHA)
  dst2da = dst_flat.reshape(NCHUNKA, CHA)
  dst2d = dst_flat.reshape(NCHUNK, CH)
  b1r = b1.reshape(1, D)
  b2r = b2.reshape(1, D)
  lw = jnp.concatenate(
      [lin_W, jnp.zeros((D, NCLS_PAD - NCLS), dtype=lin_W.dtype)], axis=1)
  lb = jnp.concatenate(
      [lin_b, jnp.full((NCLS_PAD - NCLS,), -1e30, dtype=lin_b.dtype)]
  ).reshape(1, NCLS_PAD)

  deg_kernel, agg_kernel = _sc_kernels()
  deg2 = deg_kernel(dst2d)
  hp1 = _tc1(x_pad, W1, deg2)
  acc1 = agg_kernel(hp1, src2da, dst2da)
  hp2 = _tc2(acc1, hp1, deg2, W2, b1r)
  acc2 = agg_kernel(hp2, src2da, dst2da)
  out = _tc3(acc2, hp2, deg2, b2r, lw, lb)
  return out[:N, :NCLS]


# speedup vs baseline: 25.5876x; 1.0051x over previous
import functools

import jax
import jax.numpy as jnp
from jax import lax
from jax.experimental import pallas as pl
from jax.experimental.pallas import tpu as pltpu
from jax.experimental.pallas import tpu_sc as plsc

N = 10000
E = 320000
D = 128
NCLS = 40
NCLS_PAD = 64

NC = 2
NS = 16
NW = NC * NS
CH = 128
CPW = 80
NCHUNK = NW * CPW
EPAD = NCHUNK * CH
NP = 10240
RPT = NP // NS
DUMMY = 10100
DW = 128
IB = 16

CHA = 64
NBUF = 4
CPTA = EPAD // (NW * CHA)
NCHUNKA = NW * CPTA
NBLK = CPTA // IB


def _zero_fill(buf, ncols, value=0.0):
  v = jnp.full((16,), value, dtype=jnp.float32)

  def row(i, carry):
    for g in range(ncols // 16):
      buf[i, pl.ds(g * 16, 16)] = v
    return carry

  lax.fori_loop(0, buf.shape[0], row, 0)


def _deg_body(dst_hbm, out_hbm, acc_sh, dst_v, ones_v):
  c = lax.axis_index("c")
  s = lax.axis_index("s")
  wid = s * NC + c

  pltpu.sync_copy(dst_hbm.at[pl.ds(wid * CPW, CPW)], dst_v)
  _zero_fill(ones_v, DW, 0.0)
  for k in range(RPT // CH):
    pltpu.sync_copy(ones_v, acc_sh.at[pl.ds(s * RPT + k * CH, CH)])
  _zero_fill(ones_v, DW, 1.0)
  plsc.subcore_barrier()

  def body(j, carry):
    pltpu.sync_copy(ones_v, acc_sh.at[dst_v.at[j]], add=True)
    return carry

  lax.fori_loop(0, CPW, body, 0)
  plsc.subcore_barrier()
  for k in range(RPT // CH):
    r = s * RPT + k * CH
    pltpu.sync_copy(acc_sh.at[pl.ds(r, CH)], out_hbm.at[c].at[pl.ds(r, CH)])


def _agg_body(hp_hbm, src_hbm, dst_hbm, out_hbm,
              acc_sh, src_v, dst_v, bufs, sems):
  c = lax.axis_index("c")
  s = lax.axis_index("s")
  wid = s * NC + c

  _zero_fill(bufs[0], D, 0.0)
  for k in range(RPT // CHA):
    pltpu.sync_copy(bufs[0], acc_sh.at[pl.ds(s * RPT + k * CHA, CHA)])
  plsc.subcore_barrier()

  def blk_body(bi, carry):
    base = wid * CPTA + bi * IB
    pltpu.sync_copy(src_hbm.at[pl.ds(base, IB)], src_v)
    pltpu.sync_copy(dst_hbm.at[pl.ds(base, IB)], dst_v)
    descs = [
        pltpu.async_copy(hp_hbm.at[src_v.at[j]], bufs[j], sems[j])
        for j in range(NBUF - 1)
    ]
    descs.append(None)
    for j in range(IB):
      b = j % NBUF
      descs[b].wait()
      if j + NBUF - 1 < IB:
        bn = (j + NBUF - 1) % NBUF
        descs[bn] = pltpu.async_copy(
            hp_hbm.at[src_v.at[j + NBUF - 1]], bufs[bn], sems[bn])
      pltpu.sync_copy(bufs[b], acc_sh.at[dst_v.at[j]], add=True)
    return carry

  lax.fori_loop(0, NBLK, blk_body, 0)
  plsc.subcore_barrier()
  for k in range(RPT // CH):
    r = s * RPT + k * CH
    pltpu.sync_copy(acc_sh.at[pl.ds(r, CH)], out_hbm.at[c].at[pl.ds(r, CH)])


@functools.cache
def _sc_kernels():
  mesh = plsc.VectorSubcoreMesh(
      core_axis_name="c", subcore_axis_name="s", num_cores=NC, num_subcores=NS
  )
  deg_kernel = pl.kernel(
      _deg_body,
      out_type=jax.ShapeDtypeStruct((NC, NP, DW), jnp.float32),
      mesh=mesh,
      scratch_types=[
          pltpu.VMEM_SHARED((NP, DW), jnp.float32),
          pltpu.VMEM((CPW, CH), jnp.int32),
          pltpu.VMEM((CH, DW), jnp.float32),
      ],
  )
  agg_kernel = pl.kernel(
      _agg_body,
      out_type=jax.ShapeDtypeStruct((NC, NP, D), jnp.float32),
      mesh=mesh,
      scratch_types=[
          pltpu.VMEM_SHARED((NP, D), jnp.float32),
          pltpu.VMEM((IB, CHA), jnp.int32),
          pltpu.VMEM((IB, CHA), jnp.int32),
          [pltpu.VMEM((CHA, D), jnp.float32) for _ in range(NBUF)],
          [pltpu.SemaphoreType.DMA for _ in range(NBUF)],
      ],
  )
  return deg_kernel, agg_kernel


_RB = 512
_GRID = NP // _RB


def _dinv_of(deg_ref):
  deg = deg_ref[0, :, 0:1] + deg_ref[1, :, 0:1] + 1.0
  return lax.rsqrt(deg)


def _tc_mm_body(x_ref, w1_ref, out_ref):
  out_ref[...] = jnp.dot(
      x_ref[...], w1_ref[...], preferred_element_type=jnp.float32)


def _tc_scale_body(h_ref, deg_ref, out_ref):
  out_ref[...] = h_ref[...] * _dinv_of(deg_ref)


def _tc2_body(acc_ref, hp_ref, deg_ref, w2_ref, b1_ref, out_ref):
  dinv = _dinv_of(deg_ref)
  agg = acc_ref[0] + acc_ref[1] + hp_ref[...]
  z = jnp.maximum(agg * dinv + b1_ref[...], 0.0)
  h = jnp.dot(z, w2_ref[...], preferred_element_type=jnp.float32)
  out_ref[...] = h * dinv


def _tc3_body(acc_ref, hp_ref, deg_ref, b2_ref, lw_ref, lb_ref, out_ref):
  dinv = _dinv_of(deg_ref)
  agg = acc_ref[0] + acc_ref[1] + hp_ref[...]
  z = jnp.maximum(agg * dinv + b2_ref[...], 0.0)
  logits = jnp.dot(z, lw_ref[...], preferred_element_type=jnp.float32)
  logits = logits + lb_ref[...]
  m = jnp.max(logits, axis=1, keepdims=True)
  e = jnp.exp(logits - m)
  lse = jnp.log(jnp.sum(e, axis=1, keepdims=True)) + m
  out_ref[...] = (logits - lse)[:, :NCLS]


def _row_spec():
  return pl.BlockSpec((_RB, D), lambda i: (i, 0))


def _acc_spec():
  return pl.BlockSpec((NC, _RB, D), lambda i: (0, i, 0))


def _deg_spec():
  return pl.BlockSpec((NC, _RB, DW), lambda i: (0, i, 0))


def _full(shape):
  return pl.BlockSpec(shape, lambda i: tuple(0 for _ in shape))


_tc_mm = pl.pallas_call(
    _tc_mm_body,
    grid=(_GRID,),
    in_specs=[_row_spec(), _full((D, D))],
    out_specs=_row_spec(),
    out_shape=jax.ShapeDtypeStruct((NP, D), jnp.float32),
)

_tc_scale = pl.pallas_call(
    _tc_scale_body,
    grid=(_GRID,),
    in_specs=[_row_spec(), _deg_spec()],
    out_specs=_row_spec(),
    out_shape=jax.ShapeDtypeStruct((NP, D), jnp.float32),
)

_tc2 = pl.pallas_call(
    _tc2_body,
    grid=(_GRID,),
    in_specs=[_acc_spec(), _row_spec(), _deg_spec(), _full((D, D)),
              _full((1, D))],
    out_specs=_row_spec(),
    out_shape=jax.ShapeDtypeStruct((NP, D), jnp.float32),
)

_tc3 = pl.pallas_call(
    _tc3_body,
    grid=(_GRID,),
    in_specs=[_acc_spec(), _row_spec(), _deg_spec(), _full((1, D)),
              _full((D, NCLS_PAD)), _full((1, NCLS_PAD))],
    out_specs=pl.BlockSpec((_RB, NCLS), lambda i: (i, 0)),
    out_shape=jax.ShapeDtypeStruct((N, NCLS), jnp.float32),
)


@jax.jit
def kernel(x, edge_index, W1, b1, W2, b2, lin_W, lin_b):
  x_pad = jnp.concatenate(
      [x, jnp.zeros((NP - N, D), dtype=x.dtype)], axis=0)
  pad = N + (jnp.arange(EPAD - E, dtype=jnp.int32) % (NP - N))
  src_flat = jnp.concatenate([edge_index[0], pad])
  dst_flat = jnp.concatenate([edge_index[1], pad])
  src2da = src_flat.reshape(NCHUNKA, CHA)
  dst2da = dst_flat.reshape(NCHUNKA, CHA)
  dst2d = dst_flat.reshape(NCHUNK, CH)
  b1r = b1.reshape(1, D)
  b2r = b2.reshape(1, D)
  lw = jnp.concatenate(
      [lin_W, jnp.zeros((D, NCLS_PAD - NCLS), dtype=lin_W.dtype)], axis=1)
  lb = jnp.concatenate(
      [lin_b, jnp.full((NCLS_PAD - NCLS,), -1e30, dtype=lin_b.dtype)]
  ).reshape(1, NCLS_PAD)

  deg_kernel, agg_kernel = _sc_kernels()
  deg2 = deg_kernel(dst2d)
  h1 = _tc_mm(x_pad, W1)
  hp1 = _tc_scale(h1, deg2)
  acc1 = agg_kernel(hp1, src2da, dst2da)
  hp2 = _tc2(acc1, hp1, deg2, W2, b1r)
  acc2 = agg_kernel(hp2, src2da, dst2da)
  return _tc3(acc2, hp2, deg2, b2r, lw, lb)
